# baseline (speedup 1.0000x reference)
import functools

import jax
import jax.numpy as jnp
from jax.experimental import pallas as pl
from jax.experimental.pallas import tpu as pltpu


def _rpn_body(x_ref, wk_ref, wcls_ref, wloc_ref, bconv_ref, bcls_ref,
              bloc_ref, cls_ref, loc_ref, v0_ref, v1_ref, *, n, w, hn):
    c = x_ref.shape[1]
    xb = x_ref[0].astype(jnp.bfloat16)

    pcol = jax.lax.broadcasted_iota(jnp.int32, (1, hn), 1) % w
    m_left = pcol != 0
    m_right = pcol != w - 1

    for h, v_ref in ((0, v0_ref), (1, v1_ref)):
        for k in range(9):
            dh, dw = divmod(k, 3)
            c0 = (dh - 1) * w + (dw - 1)
            lo = h * hn + c0
            ds, de = max(0, -lo), min(hn, n - lo)
            blk = xb[:, lo + ds:lo + de]
            if dw == 0:
                blk = jnp.where(m_left[:, ds:de], blk, jnp.bfloat16(0))
            elif dw == 2:
                blk = jnp.where(m_right[:, ds:de], blk, jnp.bfloat16(0))
            v_ref[k * c:(k + 1) * c, ds:de] = blk
            if ds > 0:
                v_ref[k * c:(k + 1) * c, 0:ds] = jnp.zeros((c, ds),
                                                           jnp.bfloat16)
            if de < hn:
                v_ref[k * c:(k + 1) * c, de:hn] = jnp.zeros((c, hn - de),
                                                            jnp.bfloat16)

    for h, v_ref in ((0, v0_ref), (1, v1_ref)):
        y1 = jax.lax.dot_general(
            wk_ref[...], v_ref[...],
            dimension_numbers=(((1,), (0,)), ((), ())),
            preferred_element_type=jnp.float32)
        y1 = jnp.maximum(y1 + bconv_ref[...], 0.0)
        y1 = y1.astype(jnp.bfloat16)

        cls = jax.lax.dot_general(
            wcls_ref[...], y1, dimension_numbers=(((1,), (0,)), ((), ())),
            preferred_element_type=jnp.float32) + bcls_ref[...]
        loc = jax.lax.dot_general(
            wloc_ref[...], y1, dimension_numbers=(((1,), (0,)), ((), ())),
            preferred_element_type=jnp.float32) + bloc_ref[...]

        a = cls[0:9, :]
        b = cls[9:18, :]
        m = jnp.maximum(a, b)
        ea = jnp.exp(a - m)
        eb = jnp.exp(b - m)
        denom = ea + eb
        cls_ref[0, :, h * hn:(h + 1) * hn] = jnp.concatenate(
            [ea / denom, eb / denom], axis=0)
        loc_ref[0, :, h * hn:(h + 1) * hn] = loc


def kernel(feats, gt_boxes, im_info, W_conv, b_conv, W_cls, b_cls, W_loc, b_loc):
    B, C, H, W = feats.shape
    N = H * W
    HN = N // 2
    n_cls = W_cls.shape[0]
    n_loc = W_loc.shape[0]

    x = feats.reshape(B, C, N)

    wk = W_conv.transpose(0, 2, 3, 1).reshape(C, 9 * C).astype(jnp.bfloat16)
    wcls = W_cls.reshape(n_cls, C).astype(jnp.bfloat16)
    wloc = W_loc.reshape(n_loc, C).astype(jnp.bfloat16)

    body = functools.partial(_rpn_body, n=N, w=W, hn=HN)
    cls_flat, loc_flat = pl.pallas_call(
        body,
        grid=(B,),
        in_specs=[
            pl.BlockSpec((1, C, N), lambda b: (b, 0, 0)),
            pl.BlockSpec((C, 9 * C), lambda b: (0, 0)),
            pl.BlockSpec((n_cls, C), lambda b: (0, 0)),
            pl.BlockSpec((n_loc, C), lambda b: (0, 0)),
            pl.BlockSpec((C, 1), lambda b: (0, 0)),
            pl.BlockSpec((n_cls, 1), lambda b: (0, 0)),
            pl.BlockSpec((n_loc, 1), lambda b: (0, 0)),
        ],
        out_specs=[
            pl.BlockSpec((1, n_cls, N), lambda b: (b, 0, 0)),
            pl.BlockSpec((1, n_loc, N), lambda b: (b, 0, 0)),
        ],
        out_shape=[
            jax.ShapeDtypeStruct((B, n_cls, N), jnp.float32),
            jax.ShapeDtypeStruct((B, n_loc, N), jnp.float32),
        ],
        scratch_shapes=[
            pltpu.VMEM((9 * C, HN), jnp.bfloat16),
            pltpu.VMEM((9 * C, HN), jnp.bfloat16),
        ],
        compiler_params=pltpu.CompilerParams(
            dimension_semantics=("arbitrary",)),
    )(x, wk, wcls, wloc, b_conv.reshape(C, 1), b_cls.reshape(n_cls, 1),
      b_loc.reshape(n_loc, 1))

    return (cls_flat.reshape(B, n_cls, H, W), loc_flat.reshape(B, n_loc, H, W))


# speedup vs baseline: 1.0187x; 1.0187x over previous
import functools

import jax
import jax.numpy as jnp
from jax.experimental import pallas as pl
from jax.experimental.pallas import tpu as pltpu


def _rpn_body(x_ref, wk_ref, wcls_ref, wloc_ref, bconv_ref, bcls_ref,
              bloc_ref, cls_ref, loc_ref, xm_ref, v_ref, *, n, w, margin):
    c = x_ref.shape[1]
    next_ = n + 2 * margin

    zl = jnp.zeros((c, margin), jnp.bfloat16)
    xm_ref[:, 0:margin] = zl
    xm_ref[:, margin + n:next_] = zl
    xm_ref[:, margin:margin + n] = x_ref[0].astype(jnp.bfloat16)

    pcol = jax.lax.broadcasted_iota(jnp.int32, (1, n), 1) % w
    m_left = pcol != 0
    m_right = pcol != w - 1
    for k in range(9):
        dh, dw = divmod(k, 3)
        s = margin + (dh - 1) * w + (dw - 1)
        blk = xm_ref[:, s:s + n]
        if dw == 0:
            blk = jnp.where(m_left, blk, jnp.bfloat16(0))
        elif dw == 2:
            blk = jnp.where(m_right, blk, jnp.bfloat16(0))
        v_ref[k * c:(k + 1) * c, :] = blk

    y1 = jax.lax.dot_general(
        wk_ref[...], v_ref[...],
        dimension_numbers=(((1,), (0,)), ((), ())),
        preferred_element_type=jnp.float32)
    y1 = jnp.maximum(y1 + bconv_ref[...].astype(jnp.float32), 0.0)
    y1 = y1.astype(jnp.bfloat16)

    cls = jax.lax.dot_general(
        wcls_ref[...], y1, dimension_numbers=(((1,), (0,)), ((), ())),
        preferred_element_type=jnp.float32) + bcls_ref[...]
    loc = jax.lax.dot_general(
        wloc_ref[...], y1, dimension_numbers=(((1,), (0,)), ((), ())),
        preferred_element_type=jnp.float32) + bloc_ref[...]

    a = cls[0:9, :]
    b = cls[9:18, :]
    m = jnp.maximum(a, b)
    ea = jnp.exp(a - m)
    eb = jnp.exp(b - m)
    denom = ea + eb
    cls_ref[0] = jnp.concatenate([ea / denom, eb / denom], axis=0)
    loc_ref[0] = loc


def kernel(feats, gt_boxes, im_info, W_conv, b_conv, W_cls, b_cls, W_loc, b_loc):
    B, C, H, W = feats.shape
    N = H * W
    M = W + 1
    n_cls = W_cls.shape[0]
    n_loc = W_loc.shape[0]

    x = feats.reshape(B, C, N)

    wk = W_conv.transpose(0, 2, 3, 1).reshape(C, 9 * C).astype(jnp.bfloat16)
    wcls = W_cls.reshape(n_cls, C).astype(jnp.bfloat16)
    wloc = W_loc.reshape(n_loc, C).astype(jnp.bfloat16)

    body = functools.partial(_rpn_body, n=N, w=W, margin=M)
    cls_flat, loc_flat = pl.pallas_call(
        body,
        grid=(B,),
        in_specs=[
            pl.BlockSpec((1, C, N), lambda b: (b, 0, 0)),
            pl.BlockSpec((C, 9 * C), lambda b: (0, 0)),
            pl.BlockSpec((n_cls, C), lambda b: (0, 0)),
            pl.BlockSpec((n_loc, C), lambda b: (0, 0)),
            pl.BlockSpec((C, 1), lambda b: (0, 0)),
            pl.BlockSpec((n_cls, 1), lambda b: (0, 0)),
            pl.BlockSpec((n_loc, 1), lambda b: (0, 0)),
        ],
        out_specs=[
            pl.BlockSpec((1, n_cls, N), lambda b: (b, 0, 0)),
            pl.BlockSpec((1, n_loc, N), lambda b: (b, 0, 0)),
        ],
        out_shape=[
            jax.ShapeDtypeStruct((B, n_cls, N), jnp.float32),
            jax.ShapeDtypeStruct((B, n_loc, N), jnp.float32),
        ],
        scratch_shapes=[
            pltpu.VMEM((C, N + 2 * M), jnp.bfloat16),
            pltpu.VMEM((9 * C, N), jnp.bfloat16),
        ],
        compiler_params=pltpu.CompilerParams(
            dimension_semantics=("arbitrary",)),
    )(x, wk, wcls, wloc, b_conv.astype(jnp.bfloat16).reshape(C, 1),
      b_cls.astype(jnp.float32).reshape(n_cls, 1),
      b_loc.astype(jnp.float32).reshape(n_loc, 1))

    return (cls_flat.reshape(B, n_cls, H, W), loc_flat.reshape(B, n_loc, H, W))
